# baseline (speedup 1.0000x reference)
import jax
import jax.numpy as jnp
from jax.experimental import pallas as pl
from jax.experimental.pallas import tpu as pltpu

ALPHA = 0.1
BLK = 512


def _gat_kernel(x_ref, adj_ref, w_ref, a_ref, o_ref,
                h_ref, esc_ref, etr_ref):
    i = pl.program_id(0)
    d_out = h_ref.shape[1]

    if True:
        h = jnp.dot(x_ref[...], w_ref[...], preferred_element_type=jnp.float32)
        h_ref[...] = h.astype(jnp.bfloat16)
        a1 = a_ref[0:1, :d_out]
        a2 = a_ref[0:1, d_out:]
        st = jax.lax.dot_general(
            jnp.concatenate([a1, a2], axis=0), h, (((1,), (1,)), ((), ())),
            preferred_element_type=jnp.float32)
        s_row = st[0:1, :]
        t_row = st[1:2, :]
        etr_ref[0:1, :] = jnp.exp(-t_row)
        etr_ref[1:2, :] = jnp.exp(-ALPHA * t_row)
        es = jnp.concatenate([jnp.exp(-s_row), jnp.exp(-ALPHA * s_row)], axis=0)
        esc_ref[...] = jnp.transpose(es)

    es = esc_ref[pl.ds(i * BLK, BLK), 0:1]
    es_a = esc_ref[pl.ds(i * BLK, BLK), 1:2]
    p = es * etr_ref[0:1, :]
    p_a = es_a * etr_ref[1:2, :]
    e = jnp.where(p < 1.0, p, p_a)
    e = jnp.where(adj_ref[...] != 0, e, 0.0)
    rowsum = jnp.sum(e, axis=1, keepdims=True)
    hp = jnp.dot(e.astype(jnp.bfloat16), h_ref[...],
                 preferred_element_type=jnp.float32)
    hp = hp / (rowsum + 1e-5)
    o_ref[...] = jnp.where(hp > 0, hp, jnp.exp(jnp.minimum(hp, 0.0)) - 1.0)


def kernel(input, adj, W, a):
    N, d_in = input.shape
    d_out = W.shape[1]

    grid = (N // BLK,)
    out = pl.pallas_call(
        _gat_kernel,
        grid=grid,
        in_specs=[
            pl.BlockSpec((N, d_in), lambda i: (0, 0)),
            pl.BlockSpec((BLK, N), lambda i: (i, 0)),
            pl.BlockSpec((d_in, d_out), lambda i: (0, 0)),
            pl.BlockSpec((1, 2 * d_out), lambda i: (0, 0)),
        ],
        out_specs=pl.BlockSpec((BLK, d_out), lambda i: (i, 0)),
        out_shape=jax.ShapeDtypeStruct((N, d_out), jnp.float32),
        compiler_params=pltpu.CompilerParams(dimension_semantics=("parallel",)),
        scratch_shapes=[
            pltpu.VMEM((N, d_out), jnp.bfloat16),
            pltpu.VMEM((N, 2), jnp.float32),
            pltpu.VMEM((2, N), jnp.float32),
        ],
    )(input, adj.view(jnp.int8), W, a)
    return out


# speedup vs baseline: 1.1160x; 1.1160x over previous
import jax
import jax.numpy as jnp
from jax.experimental import pallas as pl
from jax.experimental.pallas import tpu as pltpu

ALPHA = 0.1
BLK = 512


def _gat_kernel(x_ref, adj_ref, w_ref, a_ref, o_ref,
                h_ref, esc_ref, etr_ref):
    i = pl.program_id(0)
    d_out = h_ref.shape[1]

    @pl.when(i == 0)
    def _prologue():
        h = jnp.dot(x_ref[...], w_ref[...], preferred_element_type=jnp.float32)
        h_ref[...] = h.astype(jnp.bfloat16)
        a1 = a_ref[0:1, :d_out]
        a2 = a_ref[0:1, d_out:]
        st = jax.lax.dot_general(
            jnp.concatenate([a1, a2], axis=0), h, (((1,), (1,)), ((), ())),
            preferred_element_type=jnp.float32)
        s_row = st[0:1, :]
        t_row = st[1:2, :]
        etr_ref[0:1, :] = jnp.exp(-t_row)
        etr_ref[1:2, :] = jnp.exp(-ALPHA * t_row)
        es = jnp.concatenate([jnp.exp(-s_row), jnp.exp(-ALPHA * s_row)], axis=0)
        esc_ref[...] = jnp.transpose(es)

    es = esc_ref[pl.ds(i * BLK, BLK), 0:1]
    es_a = esc_ref[pl.ds(i * BLK, BLK), 1:2]
    p = es * etr_ref[0:1, :]
    p_a = es_a * etr_ref[1:2, :]
    e = jnp.minimum(p, p_a)
    e = jnp.where(adj_ref[...] != 0, e, 0.0)
    rowsum = jnp.sum(e, axis=1, keepdims=True)
    hp = jnp.dot(e.astype(jnp.bfloat16), h_ref[...],
                 preferred_element_type=jnp.float32)
    hp = hp / (rowsum + 1e-5)
    o_ref[...] = jnp.where(hp > 0, hp, jnp.exp(jnp.minimum(hp, 0.0)) - 1.0)


def kernel(input, adj, W, a):
    N, d_in = input.shape
    d_out = W.shape[1]

    grid = (N // BLK,)
    out = pl.pallas_call(
        _gat_kernel,
        grid=grid,
        in_specs=[
            pl.BlockSpec((N, d_in), lambda i: (0, 0)),
            pl.BlockSpec((BLK, N), lambda i: (i, 0)),
            pl.BlockSpec((d_in, d_out), lambda i: (0, 0)),
            pl.BlockSpec((1, 2 * d_out), lambda i: (0, 0)),
        ],
        out_specs=pl.BlockSpec((BLK, d_out), lambda i: (i, 0)),
        out_shape=jax.ShapeDtypeStruct((N, d_out), jnp.float32),
        scratch_shapes=[
            pltpu.VMEM((N, d_out), jnp.bfloat16),
            pltpu.VMEM((N, 2), jnp.float32),
            pltpu.VMEM((2, N), jnp.float32),
        ],
    )(input, adj.view(jnp.int8), W, a)
    return out


# speedup vs baseline: 1.1838x; 1.0608x over previous
import jax
import jax.numpy as jnp
from jax.experimental import pallas as pl
from jax.experimental.pallas import tpu as pltpu

ALPHA = 0.1
BLK = 512


def _gat_kernel(x_ref, adj_ref, w_ref, a_ref, o_ref,
                h_ref, esc_ref, etr_ref):
    i = pl.program_id(0)
    d_out = h_ref.shape[1]

    @pl.when(i == 0)
    def _prologue():
        h = jnp.dot(x_ref[...], w_ref[...], preferred_element_type=jnp.float32)
        h_ref[...] = h.astype(jnp.bfloat16)
        a1 = a_ref[0:1, :d_out]
        a2 = a_ref[0:1, d_out:]
        st = jax.lax.dot_general(
            jnp.concatenate([a1, a2], axis=0), h, (((1,), (1,)), ((), ())),
            preferred_element_type=jnp.float32)
        s_row = st[0:1, :]
        t_row = st[1:2, :]
        etr_ref[0:1, :] = jnp.exp(-t_row)
        etr_ref[1:2, :] = jnp.exp(-ALPHA * t_row)
        es = jnp.concatenate([jnp.exp(-s_row), jnp.exp(-ALPHA * s_row)], axis=0)
        esc_ref[...] = jnp.transpose(es)

    es = esc_ref[pl.ds(i * BLK, BLK), 0:1]
    es_a = esc_ref[pl.ds(i * BLK, BLK), 1:2]
    p = es * etr_ref[0:1, :]
    p_a = es_a * etr_ref[1:2, :]
    e = jnp.minimum(p, p_a)
    e = e * adj_ref[...].astype(jnp.float32)
    rowsum = jnp.sum(e, axis=1, keepdims=True)
    hp = jnp.dot(e.astype(jnp.bfloat16), h_ref[...],
                 preferred_element_type=jnp.float32)
    hp = hp / (rowsum + 1e-5)
    o_ref[...] = jnp.where(hp > 0, hp, jnp.exp(jnp.minimum(hp, 0.0)) - 1.0)


def kernel(input, adj, W, a):
    N, d_in = input.shape
    d_out = W.shape[1]

    grid = (N // BLK,)
    out = pl.pallas_call(
        _gat_kernel,
        grid=grid,
        in_specs=[
            pl.BlockSpec((N, d_in), lambda i: (0, 0)),
            pl.BlockSpec((BLK, N), lambda i: (i, 0)),
            pl.BlockSpec((d_in, d_out), lambda i: (0, 0)),
            pl.BlockSpec((1, 2 * d_out), lambda i: (0, 0)),
        ],
        out_specs=pl.BlockSpec((BLK, d_out), lambda i: (i, 0)),
        out_shape=jax.ShapeDtypeStruct((N, d_out), jnp.float32),
        scratch_shapes=[
            pltpu.VMEM((N, d_out), jnp.bfloat16),
            pltpu.VMEM((N, 2), jnp.float32),
            pltpu.VMEM((2, N), jnp.float32),
        ],
    )(input, adj.view(jnp.int8), W, a)
    return out
